# baseline (speedup 1.0000x reference)
import functools

import jax
import jax.numpy as jnp
from jax import lax
from jax.experimental import pallas as pl
from jax.experimental.pallas import tpu as pltpu
from jax.experimental.pallas import tpu_sc as plsc

_NC, _NS = 2, 16
_NW = _NC * _NS
_B, _K, _T = 64, 1000, 100
_KP = 1024
_NCHUNK = 1008 // 16
_BPW = _B // _NW
_BIG = 1e30


def _rsqrt_newton(x):
    i = lax.bitcast_convert_type(x, jnp.int32)
    y = lax.bitcast_convert_type(
        jnp.int32(0x5F3759DF) - lax.shift_right_arithmetic(i, 1), jnp.float32)
    for _ in range(3):
        y = y * (1.5 - 0.5 * x * y * y)
    return y


def _dyn_gather(x, idx):
    dn = lax.GatherDimensionNumbers(offset_dims=(), collapsed_slice_dims=(0,),
                                    start_index_map=(0,))
    return lax.gather(x, idx[:, None], dn, (1,),
                      mode=lax.GatherScatterMode.PROMISE_IN_BOUNDS)


def _sc_match(pred_flat, conf, tgt_flat):
    mesh = plsc.VectorSubcoreMesh(core_axis_name="c", subcore_axis_name="s")
    fkp = lambda: pltpu.VMEM((_KP,), jnp.float32)
    per_batch = [fkp(), fkp(), fkp(), fkp(), pltpu.VMEM((_K,), jnp.float32),
                 pltpu.VMEM((512,), jnp.float32),
                 pltpu.VMEM((128,), jnp.int32),
                 pltpu.VMEM((4096,), jnp.float32)]
    sems = [pltpu.SemaphoreType.DMA] * 6

    @functools.partial(
        pl.kernel,
        out_type=jax.ShapeDtypeStruct((_B, 16), jnp.float32),
        mesh=mesh,
        scratch_types=per_batch + per_batch
                      + [pltpu.VMEM((16,), jnp.float32)] + sems,
        compiler_params=pltpu.CompilerParams(needs_layout_passes=False),
    )
    def k(pred_h, conf_h, tgt_h, out_h,
          p0a, p1a, p2a, basea, confa, tgta, idxa_v, stga,
          p0b, p1b, p2b, baseb, confb, tgtb, idxb_v, stgb, rowv,
          s0, s1, s2, s3, s4, s5):
        wid = lax.axis_index("s") * _NC + lax.axis_index("c")
        iota = lax.iota(jnp.int32, 16)
        lane0 = iota == 0
        ip0 = iota * 4
        ip1 = ip0 + 1
        ip2 = ip0 + 2
        big16 = jnp.full((16,), _BIG, jnp.float32)
        zeroi = jnp.zeros((16,), jnp.int32)

        batches = ((p0a, p1a, p2a, basea, confa, tgta, idxa_v, stga),
                   (p0b, p1b, p2b, baseb, confb, tgtb, idxb_v, stgb))

        ba = wid * _BPW
        cps = (pltpu.async_copy(pred_h.at[ba], stga, s0),
               pltpu.async_copy(conf_h.at[ba], confa, s1),
               pltpu.async_copy(tgt_h.at[ba], tgta, s2),
               pltpu.async_copy(pred_h.at[ba + 1], stgb, s3),
               pltpu.async_copy(conf_h.at[ba + 1], confb, s4),
               pltpu.async_copy(tgt_h.at[ba + 1], tgtb, s5))
        for r in range(_BPW):
            p0v, p1v, p2v, basev, confv, tgtv, idxv, stagev = batches[r]
            for cp in cps[3 * r:3 * r + 3]:
                cp.wait()

            def stage_chunk(c, _):
                off = c * 16
                src = c * 64
                p0v[pl.ds(off, 16)] = plsc.load_gather(stagev, (src + ip0,))
                p1v[pl.ds(off, 16)] = plsc.load_gather(stagev, (src + ip1,))
                p2v[pl.ds(off, 16)] = plsc.load_gather(stagev, (src + ip2,))
                ci = jnp.minimum(off + iota, _K - 1)
                x = plsc.load_gather(confv, (ci,))
                basev[pl.ds(off, 16)] = -1.0 / (1.0 + jnp.exp(-x))
                return 0

            lax.fori_loop(0, _NCHUNK, stage_chunk, 0, unroll=7)
            idxv[pl.ds(96, 16)] = zeroi

        def jstep(j, carry):
            j4 = jnp.full((16,), j * 4, jnp.int32)
            ta0 = plsc.load_gather(tgta, (j4,))
            ta1 = plsc.load_gather(tgta, (j4 + 1,))
            ta2 = plsc.load_gather(tgta, (j4 + 2,))
            tb0 = plsc.load_gather(tgtb, (j4,))
            tb1 = plsc.load_gather(tgtb, (j4 + 1,))
            tb2 = plsc.load_gather(tgtb, (j4 + 2,))

            def cstep(c, cc):
                rma, ria, rmb, rib = cc
                off = c * 16
                va = (jnp.abs(p0a[pl.ds(off, 16)] - ta0)
                      + jnp.abs(p1a[pl.ds(off, 16)] - ta1)
                      + jnp.abs(p2a[pl.ds(off, 16)] - ta2)
                      ) + basea[pl.ds(off, 16)]
                vb = (jnp.abs(p0b[pl.ds(off, 16)] - tb0)
                      + jnp.abs(p1b[pl.ds(off, 16)] - tb1)
                      + jnp.abs(p2b[pl.ds(off, 16)] - tb2)
                      ) + baseb[pl.ds(off, 16)]
                ba = va < rma
                bb = vb < rmb
                rma = jnp.where(ba, va, rma)
                ria = jnp.where(ba, off + iota, ria)
                rmb = jnp.where(bb, vb, rmb)
                rib = jnp.where(bb, off + iota, rib)
                return rma, ria, rmb, rib

            rm0 = jnp.full((16,), 3e38, jnp.float32)
            ri0 = jnp.zeros((16,), jnp.int32)
            rma, ria, rmb, rib = lax.fori_loop(
                0, _NCHUNK, cstep, (rm0, ri0, rm0, ri0), unroll=9)

            j_s = jnp.full((16,), j, jnp.int32)
            mna = jnp.min(rma)
            la = plsc.all_reduce_ffs(rma == mna)
            ia = _dyn_gather(ria, jnp.broadcast_to(la, (16,)))
            plsc.store_scatter(basea, (ia,), big16, mask=lane0)
            plsc.store_scatter(idxa_v, (j_s,), ia, mask=lane0)

            mnb = jnp.min(rmb)
            lb = plsc.all_reduce_ffs(rmb == mnb)
            ib = _dyn_gather(rib, jnp.broadcast_to(lb, (16,)))
            plsc.store_scatter(baseb, (ib,), big16, mask=lane0)
            plsc.store_scatter(idxb_v, (j_s,), ib, mask=lane0)
            return 0

        lax.fori_loop(0, _T, jstep, 0)

        for r in range(_BPW):
            b = wid * _BPW + r
            p0v, p1v, p2v, _, confv, tgtv, idxv, _ = batches[r]
            dir_acc = jnp.zeros((16,), jnp.float32)
            off_acc = jnp.zeros((16,), jnp.float32)
            cm_acc = jnp.zeros((16,), jnp.float32)
            for cc in range((_T + 15) // 16):
                s = cc * 16
                iv = idxv[pl.ds(s, 16)]
                a0 = plsc.load_gather(p0v, (iv,))
                a1 = plsc.load_gather(p1v, (iv,))
                ad = plsc.load_gather(p2v, (iv,))
                cg = plsc.load_gather(confv, (iv,))
                s64 = s * 4
                b0 = plsc.load_gather(tgtv, (s64 + ip0,))
                b1 = plsc.load_gather(tgtv, (s64 + ip1,))
                bd = plsc.load_gather(tgtv, (s64 + ip2,))
                sp = jnp.maximum(a0 * a0 + a1 * a1, 1e-24)
                st = jnp.maximum(b0 * b0 + b1 * b1, 1e-24)
                rp = _rsqrt_newton(sp)
                rt = _rsqrt_newton(st)
                u = (a0 * b0 + a1 * b1) * (rp * rt)
                dirt = jnp.abs(1.0 - u)
                offt = jnp.abs(ad * rp - bd * rt)
                msk = (s + iota) < _T
                dir_acc = dir_acc + jnp.where(msk, dirt, 0.0)
                off_acc = off_acc + jnp.where(msk, offt, 0.0)
                cm_acc = cm_acc + jnp.where(msk, cg, 0.0)
            dir_s = jnp.sum(dir_acc)
            off_s = jnp.sum(off_acc)
            cm = jnp.sum(cm_acc)
            row = (jnp.where(iota == 0, dir_s, 0.0)
                   + jnp.where(iota == 1, off_s, 0.0)
                   + jnp.where(iota == 2, cm, 0.0))
            rowv[...] = row
            pltpu.sync_copy(rowv, out_h.at[b])

    return k(pred_flat, conf, tgt_flat)


def _tc_combine(conf, partials):
    def body(conf_ref, part_ref, out_ref):
        x = conf_ref[...]
        bce = jnp.sum(jnp.maximum(x, 0.0) + jnp.log1p(jnp.exp(-jnp.abs(x))))
        pr = part_ref[...]
        dir_tot = jnp.sum(pr[:, 0:1])
        off_tot = jnp.sum(pr[:, 1:2])
        cm_tot = jnp.sum(pr[:, 2:3])
        inv_bt = 1.0 / (_B * _T)
        loss_lines = (dir_tot * inv_bt + off_tot * inv_bt) * 0.5
        loss_conf = (bce - cm_tot) * (1.0 / (_B * _K))
        out_ref[0, 0] = (loss_lines + loss_conf) * 0.5

    return pl.pallas_call(
        body,
        out_shape=jax.ShapeDtypeStruct((1, 1), jnp.float32),
        out_specs=pl.BlockSpec(memory_space=pltpu.SMEM),
    )(conf, partials)


def kernel(pred_lines, confidence, targets):
    pred_flat = jnp.pad(pred_lines.reshape(_B, _K * 4), ((0, 0), (0, 96)),
                        constant_values=1e9)
    tgt_flat = jnp.pad(targets.reshape(_B, _T * 4), ((0, 0), (0, 112)))
    partials = _sc_match(pred_flat, confidence, tgt_flat)
    out = _tc_combine(confidence, partials)
    return out[0, 0]


# speedup vs baseline: 1.0189x; 1.0189x over previous
import functools

import jax
import jax.numpy as jnp
from jax import lax
from jax.experimental import pallas as pl
from jax.experimental.pallas import tpu as pltpu
from jax.experimental.pallas import tpu_sc as plsc

_NC, _NS = 2, 16
_NW = _NC * _NS
_B, _K, _T = 64, 1000, 100
_KP = 1024
_NCHUNK = 1008 // 16
_BPW = _B // _NW
_BIG = 1e30


def _rsqrt_newton(x):
    i = lax.bitcast_convert_type(x, jnp.int32)
    y = lax.bitcast_convert_type(
        jnp.int32(0x5F3759DF) - lax.shift_right_arithmetic(i, 1), jnp.float32)
    for _ in range(3):
        y = y * (1.5 - 0.5 * x * y * y)
    return y


def _dyn_gather(x, idx):
    dn = lax.GatherDimensionNumbers(offset_dims=(), collapsed_slice_dims=(0,),
                                    start_index_map=(0,))
    return lax.gather(x, idx[:, None], dn, (1,),
                      mode=lax.GatherScatterMode.PROMISE_IN_BOUNDS)


def _sc_match(pred_flat, conf, tgt_flat):
    mesh = plsc.VectorSubcoreMesh(core_axis_name="c", subcore_axis_name="s")
    fkp = lambda: pltpu.VMEM((_KP,), jnp.float32)
    per_batch = [fkp(), fkp(), fkp(), fkp(), fkp(),
                 pltpu.VMEM((512,), jnp.float32),
                 pltpu.VMEM((128,), jnp.int32),
                 pltpu.VMEM((4096,), jnp.float32)]
    sems = [pltpu.SemaphoreType.DMA] * 6

    @functools.partial(
        pl.kernel,
        out_type=jax.ShapeDtypeStruct((_B, 16), jnp.float32),
        mesh=mesh,
        scratch_types=per_batch + per_batch
                      + [pltpu.VMEM((16,), jnp.float32)] + sems,
        compiler_params=pltpu.CompilerParams(needs_layout_passes=False),
    )
    def k(pred_h, conf_h, tgt_h, out_h,
          p0a, p1a, p2a, basea, confa, tgta, idxa_v, stga,
          p0b, p1b, p2b, baseb, confb, tgtb, idxb_v, stgb, rowv,
          s0, s1, s2, s3, s4, s5):
        wid = lax.axis_index("s") * _NC + lax.axis_index("c")
        iota = lax.iota(jnp.int32, 16)
        lane0 = iota == 0
        ip0 = iota * 4
        ip1 = ip0 + 1
        ip2 = ip0 + 2
        big16 = jnp.full((16,), _BIG, jnp.float32)
        zeroi = jnp.zeros((16,), jnp.int32)

        batches = ((p0a, p1a, p2a, basea, confa, tgta, idxa_v, stga),
                   (p0b, p1b, p2b, baseb, confb, tgtb, idxb_v, stgb))

        ba = wid * _BPW
        cps = (pltpu.async_copy(pred_h.at[ba], stga, s0),
               pltpu.async_copy(conf_h.at[ba], confa, s1),
               pltpu.async_copy(tgt_h.at[ba], tgta, s2),
               pltpu.async_copy(pred_h.at[ba + 1], stgb, s3),
               pltpu.async_copy(conf_h.at[ba + 1], confb, s4),
               pltpu.async_copy(tgt_h.at[ba + 1], tgtb, s5))
        for r in range(_BPW):
            p0v, p1v, p2v, basev, confv, tgtv, idxv, stagev = batches[r]
            for cp in cps[3 * r:3 * r + 3]:
                cp.wait()

            def stage_chunk(c, _):
                off = c * 16
                src = c * 64
                p0v[pl.ds(off, 16)] = plsc.load_gather(stagev, (src + ip0,))
                p1v[pl.ds(off, 16)] = plsc.load_gather(stagev, (src + ip1,))
                p2v[pl.ds(off, 16)] = plsc.load_gather(stagev, (src + ip2,))
                x = confv[pl.ds(off, 16)]
                basev[pl.ds(off, 16)] = -1.0 / (1.0 + jnp.exp(-x))
                return 0

            lax.fori_loop(0, _NCHUNK, stage_chunk, 0, unroll=7)
            idxv[pl.ds(96, 16)] = zeroi

        def jstep(j, carry):
            j4 = jnp.full((16,), j * 4, jnp.int32)
            ta0 = plsc.load_gather(tgta, (j4,))
            ta1 = plsc.load_gather(tgta, (j4 + 1,))
            ta2 = plsc.load_gather(tgta, (j4 + 2,))
            tb0 = plsc.load_gather(tgtb, (j4,))
            tb1 = plsc.load_gather(tgtb, (j4 + 1,))
            tb2 = plsc.load_gather(tgtb, (j4 + 2,))

            def cstep(c, cc):
                rma, ria, rmb, rib = cc
                off = c * 16
                va = (jnp.abs(p0a[pl.ds(off, 16)] - ta0)
                      + jnp.abs(p1a[pl.ds(off, 16)] - ta1)
                      + jnp.abs(p2a[pl.ds(off, 16)] - ta2)
                      ) + basea[pl.ds(off, 16)]
                vb = (jnp.abs(p0b[pl.ds(off, 16)] - tb0)
                      + jnp.abs(p1b[pl.ds(off, 16)] - tb1)
                      + jnp.abs(p2b[pl.ds(off, 16)] - tb2)
                      ) + baseb[pl.ds(off, 16)]
                ba = va < rma
                bb = vb < rmb
                rma = jnp.where(ba, va, rma)
                ria = jnp.where(ba, off + iota, ria)
                rmb = jnp.where(bb, vb, rmb)
                rib = jnp.where(bb, off + iota, rib)
                return rma, ria, rmb, rib

            rm0 = jnp.full((16,), 3e38, jnp.float32)
            ri0 = jnp.zeros((16,), jnp.int32)
            rma, ria, rmb, rib = lax.fori_loop(
                0, _NCHUNK, cstep, (rm0, ri0, rm0, ri0), unroll=7)

            j_s = jnp.full((16,), j, jnp.int32)
            mna = jnp.min(rma)
            la = plsc.all_reduce_ffs(rma == mna)
            ia = _dyn_gather(ria, jnp.broadcast_to(la, (16,)))
            plsc.store_scatter(basea, (ia,), big16, mask=lane0)
            plsc.store_scatter(idxa_v, (j_s,), ia, mask=lane0)

            mnb = jnp.min(rmb)
            lb = plsc.all_reduce_ffs(rmb == mnb)
            ib = _dyn_gather(rib, jnp.broadcast_to(lb, (16,)))
            plsc.store_scatter(baseb, (ib,), big16, mask=lane0)
            plsc.store_scatter(idxb_v, (j_s,), ib, mask=lane0)
            return 0

        lax.fori_loop(0, _T, jstep, 0)

        for r in range(_BPW):
            b = wid * _BPW + r
            p0v, p1v, p2v, _, confv, tgtv, idxv, _ = batches[r]
            dir_acc = jnp.zeros((16,), jnp.float32)
            off_acc = jnp.zeros((16,), jnp.float32)
            cm_acc = jnp.zeros((16,), jnp.float32)
            for cc in range((_T + 15) // 16):
                s = cc * 16
                iv = idxv[pl.ds(s, 16)]
                a0 = plsc.load_gather(p0v, (iv,))
                a1 = plsc.load_gather(p1v, (iv,))
                ad = plsc.load_gather(p2v, (iv,))
                cg = plsc.load_gather(confv, (iv,))
                s64 = s * 4
                b0 = plsc.load_gather(tgtv, (s64 + ip0,))
                b1 = plsc.load_gather(tgtv, (s64 + ip1,))
                bd = plsc.load_gather(tgtv, (s64 + ip2,))
                sp = jnp.maximum(a0 * a0 + a1 * a1, 1e-24)
                st = jnp.maximum(b0 * b0 + b1 * b1, 1e-24)
                rp = _rsqrt_newton(sp)
                rt = _rsqrt_newton(st)
                u = (a0 * b0 + a1 * b1) * (rp * rt)
                dirt = jnp.abs(1.0 - u)
                offt = jnp.abs(ad * rp - bd * rt)
                msk = (s + iota) < _T
                dir_acc = dir_acc + jnp.where(msk, dirt, 0.0)
                off_acc = off_acc + jnp.where(msk, offt, 0.0)
                cm_acc = cm_acc + jnp.where(msk, cg, 0.0)
            dir_s = jnp.sum(dir_acc)
            off_s = jnp.sum(off_acc)
            cm = jnp.sum(cm_acc)
            row = (jnp.where(iota == 0, dir_s, 0.0)
                   + jnp.where(iota == 1, off_s, 0.0)
                   + jnp.where(iota == 2, cm, 0.0))
            rowv[...] = row
            pltpu.sync_copy(rowv, out_h.at[b])

    return k(pred_flat, conf, tgt_flat)


def _tc_combine(conf, partials):
    def body(conf_ref, part_ref, out_ref):
        x = conf_ref[...]
        bce = jnp.sum(jnp.maximum(x, 0.0) + jnp.log1p(jnp.exp(-jnp.abs(x))))
        pr = part_ref[...]
        dir_tot = jnp.sum(pr[:, 0:1])
        off_tot = jnp.sum(pr[:, 1:2])
        cm_tot = jnp.sum(pr[:, 2:3])
        inv_bt = 1.0 / (_B * _T)
        loss_lines = (dir_tot * inv_bt + off_tot * inv_bt) * 0.5
        loss_conf = (bce - cm_tot) * (1.0 / (_B * _K))
        out_ref[0, 0] = (loss_lines + loss_conf) * 0.5

    return pl.pallas_call(
        body,
        out_shape=jax.ShapeDtypeStruct((1, 1), jnp.float32),
        out_specs=pl.BlockSpec(memory_space=pltpu.SMEM),
    )(conf, partials)


def kernel(pred_lines, confidence, targets):
    pred_flat = jnp.pad(pred_lines.reshape(_B, _K * 4), ((0, 0), (0, 96)),
                        constant_values=1e9)
    conf_pad = jnp.pad(confidence, ((0, 0), (0, _KP - _K)))
    tgt_flat = jnp.pad(targets.reshape(_B, _T * 4), ((0, 0), (0, 112)))
    partials = _sc_match(pred_flat, conf_pad, tgt_flat)
    out = _tc_combine(confidence, partials)
    return out[0, 0]


# speedup vs baseline: 1.0248x; 1.0058x over previous
import functools

import jax
import jax.numpy as jnp
from jax import lax
from jax.experimental import pallas as pl
from jax.experimental.pallas import tpu as pltpu
from jax.experimental.pallas import tpu_sc as plsc

_NC, _NS = 2, 16
_NW = _NC * _NS
_B, _K, _T = 64, 1000, 100
_KP = 1024
_NCHUNK = 1008 // 16
_BPW = _B // _NW
_BIG = 1e30


def _rsqrt_newton(x):
    i = lax.bitcast_convert_type(x, jnp.int32)
    y = lax.bitcast_convert_type(
        jnp.int32(0x5F3759DF) - lax.shift_right_arithmetic(i, 1), jnp.float32)
    for _ in range(3):
        y = y * (1.5 - 0.5 * x * y * y)
    return y


def _dyn_gather(x, idx):
    dn = lax.GatherDimensionNumbers(offset_dims=(), collapsed_slice_dims=(0,),
                                    start_index_map=(0,))
    return lax.gather(x, idx[:, None], dn, (1,),
                      mode=lax.GatherScatterMode.PROMISE_IN_BOUNDS)


def _sc_match(pred_flat, conf, tgt_flat):
    mesh = plsc.VectorSubcoreMesh(core_axis_name="c", subcore_axis_name="s")
    fkp = lambda: pltpu.VMEM((_KP,), jnp.float32)
    per_batch = [fkp(), fkp(), fkp(), fkp(), fkp(),
                 pltpu.VMEM((512,), jnp.float32),
                 pltpu.VMEM((128,), jnp.int32),
                 pltpu.VMEM((4096,), jnp.float32)]
    sems = [pltpu.SemaphoreType.DMA] * 6

    @functools.partial(
        pl.kernel,
        out_type=jax.ShapeDtypeStruct((_B, 16), jnp.float32),
        mesh=mesh,
        scratch_types=per_batch + per_batch
                      + [pltpu.VMEM((16,), jnp.float32)] + sems,
        compiler_params=pltpu.CompilerParams(needs_layout_passes=False),
    )
    def k(pred_h, conf_h, tgt_h, out_h,
          p0a, p1a, p2a, basea, confa, tgta, idxa_v, stga,
          p0b, p1b, p2b, baseb, confb, tgtb, idxb_v, stgb, rowv,
          s0, s1, s2, s3, s4, s5):
        wid = lax.axis_index("s") * _NC + lax.axis_index("c")
        iota = lax.iota(jnp.int32, 16)
        lane0 = iota == 0
        ip0 = iota * 4
        ip1 = ip0 + 1
        ip2 = ip0 + 2
        big16 = jnp.full((16,), _BIG, jnp.float32)
        zeroi = jnp.zeros((16,), jnp.int32)

        batches = ((p0a, p1a, p2a, basea, confa, tgta, idxa_v, stga),
                   (p0b, p1b, p2b, baseb, confb, tgtb, idxb_v, stgb))

        ba = wid * _BPW
        cps = (pltpu.async_copy(pred_h.at[ba], stga, s0),
               pltpu.async_copy(conf_h.at[ba], confa, s1),
               pltpu.async_copy(tgt_h.at[ba], tgta, s2),
               pltpu.async_copy(pred_h.at[ba + 1], stgb, s3),
               pltpu.async_copy(conf_h.at[ba + 1], confb, s4),
               pltpu.async_copy(tgt_h.at[ba + 1], tgtb, s5))
        for r in range(_BPW):
            p0v, p1v, p2v, basev, confv, tgtv, idxv, stagev = batches[r]
            for cp in cps[3 * r:3 * r + 3]:
                cp.wait()

            def stage_chunk(c, _):
                off = c * 16
                src = c * 64
                p0v[pl.ds(off, 16)] = plsc.load_gather(stagev, (src + ip0,))
                p1v[pl.ds(off, 16)] = plsc.load_gather(stagev, (src + ip1,))
                p2v[pl.ds(off, 16)] = plsc.load_gather(stagev, (src + ip2,))
                x = confv[pl.ds(off, 16)]
                basev[pl.ds(off, 16)] = -1.0 / (1.0 + jnp.exp(-x))
                return 0

            lax.fori_loop(0, _NCHUNK, stage_chunk, 0, unroll=7)
            idxv[pl.ds(96, 16)] = zeroi

        def jstep(j, carry):
            j4 = jnp.full((16,), j * 4, jnp.int32)
            ta0 = plsc.load_gather(tgta, (j4,))
            ta1 = plsc.load_gather(tgta, (j4 + 1,))
            ta2 = plsc.load_gather(tgta, (j4 + 2,))
            tb0 = plsc.load_gather(tgtb, (j4,))
            tb1 = plsc.load_gather(tgtb, (j4 + 1,))
            tb2 = plsc.load_gather(tgtb, (j4 + 2,))

            def cstep(c, cc):
                rma, ria, rmb, rib = cc
                off = c * 16
                va = (jnp.abs(p0a[pl.ds(off, 16)] - ta0)
                      + jnp.abs(p1a[pl.ds(off, 16)] - ta1)
                      + jnp.abs(p2a[pl.ds(off, 16)] - ta2)
                      ) + basea[pl.ds(off, 16)]
                vb = (jnp.abs(p0b[pl.ds(off, 16)] - tb0)
                      + jnp.abs(p1b[pl.ds(off, 16)] - tb1)
                      + jnp.abs(p2b[pl.ds(off, 16)] - tb2)
                      ) + baseb[pl.ds(off, 16)]
                ba = va < rma
                bb = vb < rmb
                rma = jnp.where(ba, va, rma)
                ria = jnp.where(ba, off + iota, ria)
                rmb = jnp.where(bb, vb, rmb)
                rib = jnp.where(bb, off + iota, rib)
                return rma, ria, rmb, rib

            rm0 = jnp.full((16,), 3e38, jnp.float32)
            ri0 = jnp.zeros((16,), jnp.int32)
            rma, ria, rmb, rib = lax.fori_loop(
                0, _NCHUNK, cstep, (rm0, ri0, rm0, ri0), unroll=9)

            j_s = jnp.full((16,), j, jnp.int32)
            mna = jnp.min(rma)
            la = plsc.all_reduce_ffs(rma == mna)
            ia = _dyn_gather(ria, jnp.broadcast_to(la, (16,)))
            plsc.store_scatter(basea, (ia,), big16, mask=lane0)
            plsc.store_scatter(idxa_v, (j_s,), ia, mask=lane0)

            mnb = jnp.min(rmb)
            lb = plsc.all_reduce_ffs(rmb == mnb)
            ib = _dyn_gather(rib, jnp.broadcast_to(lb, (16,)))
            plsc.store_scatter(baseb, (ib,), big16, mask=lane0)
            plsc.store_scatter(idxb_v, (j_s,), ib, mask=lane0)
            return 0

        lax.fori_loop(0, _T, jstep, 0)

        for r in range(_BPW):
            b = wid * _BPW + r
            p0v, p1v, p2v, _, confv, tgtv, idxv, _ = batches[r]
            dir_acc = jnp.zeros((16,), jnp.float32)
            off_acc = jnp.zeros((16,), jnp.float32)
            cm_acc = jnp.zeros((16,), jnp.float32)
            for cc in range((_T + 15) // 16):
                s = cc * 16
                iv = idxv[pl.ds(s, 16)]
                a0 = plsc.load_gather(p0v, (iv,))
                a1 = plsc.load_gather(p1v, (iv,))
                ad = plsc.load_gather(p2v, (iv,))
                cg = plsc.load_gather(confv, (iv,))
                s64 = s * 4
                b0 = plsc.load_gather(tgtv, (s64 + ip0,))
                b1 = plsc.load_gather(tgtv, (s64 + ip1,))
                bd = plsc.load_gather(tgtv, (s64 + ip2,))
                sp = jnp.maximum(a0 * a0 + a1 * a1, 1e-24)
                st = jnp.maximum(b0 * b0 + b1 * b1, 1e-24)
                rp = _rsqrt_newton(sp)
                rt = _rsqrt_newton(st)
                u = (a0 * b0 + a1 * b1) * (rp * rt)
                dirt = jnp.abs(1.0 - u)
                offt = jnp.abs(ad * rp - bd * rt)
                msk = (s + iota) < _T
                dir_acc = dir_acc + jnp.where(msk, dirt, 0.0)
                off_acc = off_acc + jnp.where(msk, offt, 0.0)
                cm_acc = cm_acc + jnp.where(msk, cg, 0.0)
            dir_s = jnp.sum(dir_acc)
            off_s = jnp.sum(off_acc)
            cm = jnp.sum(cm_acc)
            row = (jnp.where(iota == 0, dir_s, 0.0)
                   + jnp.where(iota == 1, off_s, 0.0)
                   + jnp.where(iota == 2, cm, 0.0))
            rowv[...] = row
            pltpu.sync_copy(rowv, out_h.at[b])

    return k(pred_flat, conf, tgt_flat)


def _tc_combine(conf, partials):
    def body(conf_ref, part_ref, out_ref):
        x = conf_ref[...]
        bce = jnp.sum(jnp.maximum(x, 0.0) + jnp.log1p(jnp.exp(-jnp.abs(x))))
        pr = part_ref[...]
        dir_tot = jnp.sum(pr[:, 0:1])
        off_tot = jnp.sum(pr[:, 1:2])
        cm_tot = jnp.sum(pr[:, 2:3])
        inv_bt = 1.0 / (_B * _T)
        loss_lines = (dir_tot * inv_bt + off_tot * inv_bt) * 0.5
        loss_conf = (bce - cm_tot) * (1.0 / (_B * _K))
        out_ref[0, 0] = (loss_lines + loss_conf) * 0.5

    return pl.pallas_call(
        body,
        out_shape=jax.ShapeDtypeStruct((1, 1), jnp.float32),
        out_specs=pl.BlockSpec(memory_space=pltpu.SMEM),
    )(conf, partials)


def kernel(pred_lines, confidence, targets):
    pred_flat = jnp.pad(pred_lines.reshape(_B, _K * 4), ((0, 0), (0, 96)),
                        constant_values=1e9)
    conf_pad = jnp.pad(confidence, ((0, 0), (0, _KP - _K)))
    tgt_flat = jnp.pad(targets.reshape(_B, _T * 4), ((0, 0), (0, 112)))
    partials = _sc_match(pred_flat, conf_pad, tgt_flat)
    out = _tc_combine(confidence, partials)
    return out[0, 0]
